# baseline (speedup 1.0000x reference)
import functools

import jax
import jax.numpy as jnp
from jax.experimental import pallas as pl
from jax.experimental.pallas import tpu as pltpu


def _stage_weights(w1_ref, w2_ref, w3_ref, wg_ref, w1c, w2bd, w3bd,
                   n_task, n_exp):
    d = w1_ref.shape[1]
    h1 = w1_ref.shape[2]
    h2 = w2_ref.shape[2]
    odim = w3_ref.shape[2]
    for g in range(n_exp // 2):
        pair = jnp.concatenate(
            [w1_ref[2 * g], w1_ref[2 * g + 1]], axis=1)
        w1c[:, 2 * g * h1:(2 * g + 2) * h1] = pair.astype(jnp.bfloat16)
    del d
    w1c[:, n_exp * h1:] = wg_ref[...].astype(jnp.bfloat16)

    w2f = w2_ref[...].reshape(n_exp * h1, h2)
    w2t = jnp.concatenate([w2f] * n_exp, axis=1)
    r2 = jax.lax.broadcasted_iota(jnp.int32, w2t.shape, 0) // h1
    c2 = jax.lax.broadcasted_iota(jnp.int32, w2t.shape, 1) // h2
    w2bd[...] = jnp.where(r2 == c2, w2t, 0.0).astype(jnp.bfloat16)

    w3f = w3_ref[...].reshape(n_exp * h2, odim)
    w3t = jnp.concatenate([w3f] * n_exp, axis=1)
    r3 = jax.lax.broadcasted_iota(jnp.int32, w3t.shape, 0) // h2
    c3 = jax.lax.broadcasted_iota(jnp.int32, w3t.shape, 1) // odim
    w3bd[...] = jnp.where(r3 == c3, w3t, 0.0).astype(jnp.bfloat16)


def _stage_bias_row(b_ref, brow, n_rows, width):
    per = max(1, 128 // width)
    for g in range(0, n_rows, per):
        k = min(per, n_rows - g)
        seg = jnp.concatenate(
            [b_ref[pl.ds(g + j, 1), :] for j in range(k)], axis=1)
        brow[:, g * width:g * width + k * width] = seg


def _cv_sq(v, n):
    m = jnp.sum(v, axis=1, keepdims=True) / n
    var = jnp.sum((v - m) ** 2, axis=1, keepdims=True) / (n - 1)
    return var / (m * m + 1e-10)


def _moe_kernel(x_ref, w1_ref, b1_ref, w2_ref, b2_ref, w3_ref, b3_ref,
                wg_ref, bg_ref, out_ref, loss_ref, w1c, w2bd, w3bd,
                brow1, brow2, brow3, bgrow, stats, *, n_task, n_exp):
    tb = x_ref.shape[0]
    h1dim = w1_ref.shape[2]
    h2dim = w2_ref.shape[2]
    outdim = w3_ref.shape[2]
    nh1 = n_exp * h1dim

    @pl.when(pl.program_id(0) == 0)
    def _prep():
        _stage_weights(w1_ref, w2_ref, w3_ref, wg_ref, w1c, w2bd, w3bd,
                       n_task, n_exp)
        _stage_bias_row(b1_ref, brow1, n_exp, h1dim)
        _stage_bias_row(b2_ref, brow2, n_exp, h2dim)
        _stage_bias_row(b3_ref, brow3, n_exp, outdim)
        _stage_bias_row(bg_ref, bgrow, n_task, n_exp)
        stats[...] = jnp.zeros_like(stats)

    rows = tb // 2 if tb % 2 == 0 else tb
    rsel = jax.lax.broadcasted_iota(jnp.int32, (n_exp, n_exp * outdim), 0)
    csel = jax.lax.broadcasted_iota(jnp.int32, (n_exp, n_exp * outdim), 1)
    srep = (rsel == csel // outdim).astype(jnp.bfloat16)

    for r0 in range(0, tb, rows):
        xb = x_ref[pl.ds(r0, rows), :].astype(jnp.bfloat16)

        h1l = jnp.dot(xb, w1c[...], preferred_element_type=jnp.float32)
        h1 = jnp.maximum(h1l[:, :nh1] + brow1[...], 0.0)
        logits = h1l[:, nh1:] + bgrow[...]

        gates = []
        for t in range(n_task):
            lt = logits[:, t * n_exp:(t + 1) * n_exp]
            m1 = jnp.max(lt, axis=1, keepdims=True)
            sel1 = lt == m1
            masked = jnp.where(sel1, -jnp.inf, lt)
            m2 = jnp.max(masked, axis=1, keepdims=True)
            sel2 = masked == m2
            z = jnp.exp(m2 - m1)
            g1 = 1.0 / (1.0 + z)
            g2 = z / (1.0 + z)
            gates.append(jnp.where(sel1, g1, 0.0) +
                         jnp.where(sel2, g2, 0.0))

        h2 = jnp.dot(h1.astype(jnp.bfloat16), w2bd[...],
                     preferred_element_type=jnp.float32)
        h2 = jnp.maximum(h2 + brow2[...], 0.0)
        h3 = jnp.dot(h2.astype(jnp.bfloat16), w3bd[...],
                     preferred_element_type=jnp.float32)
        h3 = jnp.maximum(h3 + brow3[...], 0.0)

        for t in range(n_task):
            grep = jnp.dot(gates[t].astype(jnp.bfloat16), srep,
                           preferred_element_type=jnp.float32)
            prod = grep * h3
            acc = prod[:, 0:outdim]
            for e in range(1, n_exp):
                acc = acc + prod[:, e * outdim:(e + 1) * outdim]
            out_ref[t, pl.ds(r0, rows), :] = acc

        imp = jnp.concatenate(
            [jnp.sum(g, axis=0, keepdims=True) for g in gates], axis=0)
        load = jnp.concatenate(
            [jnp.sum((g > 0.0).astype(jnp.float32), axis=0, keepdims=True)
             for g in gates], axis=0)
        stats[...] += jnp.concatenate(
            [imp, load,
             jnp.zeros((8 - 2 * n_task, imp.shape[1]), jnp.float32)], axis=0)

    @pl.when(pl.program_id(0) == pl.num_programs(0) - 1)
    def _loss():
        total = jnp.zeros((1, 1), jnp.float32)
        for t in range(n_task):
            cvi = _cv_sq(stats[pl.ds(t, 1), :], n_exp)
            cvl = _cv_sq(stats[pl.ds(n_task + t, 1), :], n_exp)
            total = total + (cvi + cvl) * 0.01
        loss_ref[...] = total


@functools.partial(jax.jit, static_argnames=())
def kernel(x, W1, b1, W2, b2, W3, b3, wg, bg):
    B, D = x.shape
    E, _, H1 = W1.shape
    T = wg.shape[0]
    OUT = W3.shape[2]
    H2 = W2.shape[2]
    TB = 1024 if B % 1024 == 0 else B
    grid = (B // TB,)

    _run = pl.pallas_call(
        functools.partial(_moe_kernel, n_task=T, n_exp=E),
        grid=grid,
        in_specs=[
            pl.BlockSpec((TB, D), lambda i: (i, 0)),
            pl.BlockSpec((E, D, H1), lambda i: (0, 0, 0)),
            pl.BlockSpec((E, H1), lambda i: (0, 0)),
            pl.BlockSpec((E, H1, H2), lambda i: (0, 0, 0)),
            pl.BlockSpec((E, H2), lambda i: (0, 0)),
            pl.BlockSpec((E, H2, OUT), lambda i: (0, 0, 0)),
            pl.BlockSpec((E, OUT), lambda i: (0, 0)),
            pl.BlockSpec((D, T * E), lambda i: (0, 0)),
            pl.BlockSpec((T, E), lambda i: (0, 0)),
        ],
        out_specs=[
            pl.BlockSpec((T, TB, OUT), lambda i: (0, i, 0)),
            pl.BlockSpec((1, 1), lambda i: (0, 0)),
        ],
        out_shape=[
            jax.ShapeDtypeStruct((T, B, OUT), jnp.float32),
            jax.ShapeDtypeStruct((1, 1), jnp.float32),
        ],
        scratch_shapes=[
            pltpu.VMEM((D, E * H1 + T * E), jnp.bfloat16),
            pltpu.VMEM((E * H1, E * H2), jnp.bfloat16),
            pltpu.VMEM((E * H2, E * OUT), jnp.bfloat16),
            pltpu.VMEM((1, E * H1), jnp.float32),
            pltpu.VMEM((1, E * H2), jnp.float32),
            pltpu.VMEM((1, E * OUT), jnp.float32),
            pltpu.VMEM((1, T * E), jnp.float32),
            pltpu.VMEM((8, E), jnp.float32),
        ],
        compiler_params=pltpu.CompilerParams(
            dimension_semantics=("arbitrary",),
            vmem_limit_bytes=67000000),
    )
    wgc = jnp.concatenate([wg[t] for t in range(T)], axis=1)
    out, loss = _run(x, W1, b1, W2, b2, W3, b3, wgc, bg)
    return out, loss.reshape(())


# speedup vs baseline: 1.0542x; 1.0542x over previous
import functools

import jax
import jax.numpy as jnp
from jax.experimental import pallas as pl
from jax.experimental.pallas import tpu as pltpu


def _stage_weights(w1_ref, w2_ref, w3_ref, wg_ref, w1c, w2bd, w3bd,
                   n_task, n_exp):
    d = w1_ref.shape[1]
    h1 = w1_ref.shape[2]
    h2 = w2_ref.shape[2]
    odim = w3_ref.shape[2]
    for g in range(n_exp // 2):
        pair = jnp.concatenate(
            [w1_ref[2 * g], w1_ref[2 * g + 1]], axis=1)
        w1c[:, 2 * g * h1:(2 * g + 2) * h1] = pair
    del d
    gcols = jnp.concatenate([wg_ref[t] for t in range(n_task)], axis=1)
    w1c[:, n_exp * h1:] = gcols

    w2f = w2_ref[...].reshape(n_exp * h1, h2)
    w2t = jnp.concatenate([w2f] * n_exp, axis=1)
    r2 = jax.lax.broadcasted_iota(jnp.int32, w2t.shape, 0) // h1
    c2 = jax.lax.broadcasted_iota(jnp.int32, w2t.shape, 1) // h2
    w2bd[...] = jnp.where(r2 == c2, w2t, 0.0).astype(jnp.bfloat16)

    w3f = w3_ref[...].reshape(n_exp * h2, odim)
    w3t = jnp.concatenate([w3f] * n_exp, axis=1)
    r3 = jax.lax.broadcasted_iota(jnp.int32, w3t.shape, 0) // h2
    c3 = jax.lax.broadcasted_iota(jnp.int32, w3t.shape, 1) // odim
    w3bd[...] = jnp.where(r3 == c3, w3t, 0.0).astype(jnp.bfloat16)


def _stage_bias_row(b_ref, brow, n_rows, width):
    per = max(1, 128 // width)
    for g in range(0, n_rows, per):
        k = min(per, n_rows - g)
        seg = jnp.concatenate(
            [b_ref[pl.ds(g + j, 1), :] for j in range(k)], axis=1)
        brow[:, g * width:g * width + k * width] = seg


def _cv_sq(v, n):
    m = jnp.sum(v, axis=1, keepdims=True) / n
    var = jnp.sum((v - m) ** 2, axis=1, keepdims=True) / (n - 1)
    return var / (m * m + 1e-10)


def _moe_kernel(x_ref, w1_ref, b1_ref, w2_ref, b2_ref, w3_ref, b3_ref,
                wg_ref, bg_ref, out_ref, loss_ref, w1c, w2bd, w3bd,
                brow1, brow2, brow3, bgrow, stats, *, n_task, n_exp):
    tb = x_ref.shape[0]
    h1dim = w1_ref.shape[2]
    h2dim = w2_ref.shape[2]
    outdim = w3_ref.shape[2]
    nh1 = n_exp * h1dim

    @pl.when(pl.program_id(0) == 0)
    def _prep():
        _stage_weights(w1_ref, w2_ref, w3_ref, wg_ref, w1c, w2bd, w3bd,
                       n_task, n_exp)
        _stage_bias_row(b1_ref, brow1, n_exp, h1dim)
        _stage_bias_row(b2_ref, brow2, n_exp, h2dim)
        _stage_bias_row(b3_ref, brow3, n_exp, outdim)
        _stage_bias_row(bg_ref, bgrow, n_task, n_exp)
        stats[...] = jnp.zeros_like(stats)

    rows = tb // 2 if tb % 2 == 0 else tb
    rsel = jax.lax.broadcasted_iota(jnp.int32, (n_exp, n_exp * outdim), 0)
    csel = jax.lax.broadcasted_iota(jnp.int32, (n_exp, n_exp * outdim), 1)
    srep = (rsel == csel // outdim).astype(jnp.bfloat16)

    for r0 in range(0, tb, rows):
        xb = x_ref[pl.ds(r0, rows), :].astype(jnp.bfloat16)

        h1l = jnp.dot(xb, w1c[...], preferred_element_type=jnp.float32)
        h1 = jnp.maximum(h1l[:, :nh1] + brow1[...], 0.0)
        logits = h1l[:, nh1:] + bgrow[...]

        gates = []
        for t in range(n_task):
            lt = logits[:, t * n_exp:(t + 1) * n_exp]
            m1 = jnp.max(lt, axis=1, keepdims=True)
            sel1 = lt == m1
            masked = jnp.where(sel1, -jnp.inf, lt)
            m2 = jnp.max(masked, axis=1, keepdims=True)
            sel2 = masked == m2
            z = jnp.exp(m2 - m1)
            g1 = 1.0 / (1.0 + z)
            g2 = z / (1.0 + z)
            gates.append(jnp.where(sel1, g1, 0.0) +
                         jnp.where(sel2, g2, 0.0))

        h2 = jnp.dot(h1.astype(jnp.bfloat16), w2bd[...],
                     preferred_element_type=jnp.float32)
        h2 = jnp.maximum(h2 + brow2[...], 0.0)
        h3 = jnp.dot(h2.astype(jnp.bfloat16), w3bd[...],
                     preferred_element_type=jnp.float32)
        h3 = jnp.maximum(h3 + brow3[...], 0.0)

        for t in range(n_task):
            grep = jnp.dot(gates[t].astype(jnp.bfloat16), srep,
                           preferred_element_type=jnp.float32)
            prod = grep * h3
            acc = prod[:, 0:outdim]
            for e in range(1, n_exp):
                acc = acc + prod[:, e * outdim:(e + 1) * outdim]
            out_ref[t, pl.ds(r0, rows), :] = acc

        imp = jnp.concatenate(
            [jnp.sum(g, axis=0, keepdims=True) for g in gates], axis=0)
        load = jnp.concatenate(
            [jnp.sum((g > 0.0).astype(jnp.float32), axis=0, keepdims=True)
             for g in gates], axis=0)
        stats[...] += jnp.concatenate(
            [imp, load,
             jnp.zeros((8 - 2 * n_task, imp.shape[1]), jnp.float32)], axis=0)

    @pl.when(pl.program_id(0) == pl.num_programs(0) - 1)
    def _loss():
        total = jnp.zeros((1, 1), jnp.float32)
        for t in range(n_task):
            cvi = _cv_sq(stats[pl.ds(t, 1), :], n_exp)
            cvl = _cv_sq(stats[pl.ds(n_task + t, 1), :], n_exp)
            total = total + (cvi + cvl) * 0.01
        loss_ref[...] = total


@functools.partial(jax.jit, static_argnames=())
def kernel(x, W1, b1, W2, b2, W3, b3, wg, bg):
    B, D = x.shape
    E, _, H1 = W1.shape
    T = wg.shape[0]
    OUT = W3.shape[2]
    H2 = W2.shape[2]
    TB = 1024 if B % 1024 == 0 else B
    grid = (B // TB,)

    out, loss = pl.pallas_call(
        functools.partial(_moe_kernel, n_task=T, n_exp=E),
        grid=grid,
        in_specs=[
            pl.BlockSpec((TB, D), lambda i: (i, 0)),
            pl.BlockSpec((E, D, H1), lambda i: (0, 0, 0)),
            pl.BlockSpec((E, H1), lambda i: (0, 0)),
            pl.BlockSpec((E, H1, H2), lambda i: (0, 0, 0)),
            pl.BlockSpec((E, H2), lambda i: (0, 0)),
            pl.BlockSpec((E, H2, OUT), lambda i: (0, 0, 0)),
            pl.BlockSpec((E, OUT), lambda i: (0, 0)),
            pl.BlockSpec((T, D, E), lambda i: (0, 0, 0)),
            pl.BlockSpec((T, E), lambda i: (0, 0)),
        ],
        out_specs=[
            pl.BlockSpec((T, TB, OUT), lambda i: (0, i, 0)),
            pl.BlockSpec((1, 1), lambda i: (0, 0)),
        ],
        out_shape=[
            jax.ShapeDtypeStruct((T, B, OUT), jnp.float32),
            jax.ShapeDtypeStruct((1, 1), jnp.float32),
        ],
        scratch_shapes=[
            pltpu.VMEM((D, E * H1 + T * E), jnp.bfloat16),
            pltpu.VMEM((E * H1, E * H2), jnp.bfloat16),
            pltpu.VMEM((E * H2, E * OUT), jnp.bfloat16),
            pltpu.VMEM((1, E * H1), jnp.float32),
            pltpu.VMEM((1, E * H2), jnp.float32),
            pltpu.VMEM((1, E * OUT), jnp.float32),
            pltpu.VMEM((1, T * E), jnp.float32),
            pltpu.VMEM((8, E), jnp.float32),
        ],
        compiler_params=pltpu.CompilerParams(
            dimension_semantics=("arbitrary",)),
    )(x, W1.astype(jnp.bfloat16), b1, W2, b2, W3, b3,
      wg.astype(jnp.bfloat16), bg)

    return out, loss.reshape(())
